# baseline (speedup 1.0000x reference)
import functools

import jax
import jax.numpy as jnp
from jax import lax
from jax.experimental import pallas as pl
from jax.experimental.pallas import tpu as pltpu
from jax.experimental.pallas import tpu_sc as plsc

_MU = 0.02
_MU2 = _MU * _MU
_NW = 32
_ROWS = 16
_SCP = 5
_PERW = _SCP * 512 // _NW
_NCH = _PERW // _ROWS
_U = 4
_TROWS = 256


def _sc_body(pred_hbm, tgt_hbm, out_hbm,
             bpa, b0a, b1a, bpb, b0b, b1b, acc, sema, semb):
    c = lax.axis_index("c")
    s = lax.axis_index("s")
    wid = s * 2 + c

    def zero_body(j, carry):
        acc[pl.ds(j * 16, 16)] = jnp.zeros((16,), jnp.float32)
        return carry

    lax.fori_loop(0, 32, zero_body, 0)

    lane = lax.iota(jnp.int32, 16)
    ones = jnp.ones((16,), jnp.float32)

    def start(ck, bp, b0, b1, sem):
        gr = wid * _PERW + ck * _ROWS
        bat = gr // 512
        r0 = gr % 512
        pltpu.async_copy(pred_hbm.at[bat, 0, pl.ds(r0, _ROWS), :], bp, sem)
        pltpu.async_copy(tgt_hbm.at[bat, 0, pl.ds(r0, _ROWS), :], b0, sem)
        pltpu.async_copy(tgt_hbm.at[bat, 1, pl.ds(r0, _ROWS), :], b1, sem)

    def drain(bp, b0, b1, sem):
        pltpu.make_async_copy(pred_hbm.at[0, 0, pl.ds(0, _ROWS), :], bp, sem).wait()
        pltpu.make_async_copy(pred_hbm.at[0, 0, pl.ds(0, _ROWS), :], b0, sem).wait()
        pltpu.make_async_copy(pred_hbm.at[0, 0, pl.ds(0, _ROWS), :], b1, sem).wait()

    def compute(bp, b0, b1):
        def vec_body(j, c2):
            row = j >> 3
            col = (j & 7) * (16 * _U)
            ps, as_, ts = [], [], []
            for u in range(_U):
                o = col + u * 16
                ps.append(bp[row, pl.ds(o, 16)])
                as_.append(b0[row, pl.ds(o, 16)])
                ts.append(b1[row, pl.ds(o, 16)])
            lts, idxs = [], []
            for u in range(_U):
                p, a, t = ps[u], as_[u], ts[u]
                d = (p - a) / (a + 1e-4)
                x = d * d + _MU2
                i = plsc.bitcast(x, jnp.int32)
                y = plsc.bitcast(0x5F3759DF - (i >> 1), jnp.float32)
                xh = 0.5 * x
                y = y * (1.5 - xh * y * y)
                y = y * (1.5 - xh * y * y)
                lts.append((x * y - _MU) * t)
                g = jnp.abs(d) * y
                b = jnp.minimum(g * 10.0, 9.0).astype(jnp.int32)
                idxs.append(b * 16 + lane)
            for u in range(_U):
                plsc.addupdate_scatter(acc, [idxs[u]], lts[u])
                plsc.addupdate_scatter(acc, [idxs[u] + 256], ones,
                                       mask=ts[u] > 0)
            return c2

        lax.fori_loop(0, _ROWS * 512 // (16 * _U), vec_body, 0)

    bufs = ((bpa, b0a, b1a, sema), (bpb, b0b, b1b, semb))
    start(0, *bufs[0])
    for k in range(_NCH):
        cur = bufs[k % 2]
        if k + 1 < _NCH:
            start(k + 1, *bufs[(k + 1) % 2])
        drain(*cur)
        compute(cur[0], cur[1], cur[2])
    pltpu.sync_copy(acc, out_hbm.at[wid])


@functools.lru_cache(maxsize=1)
def _sc_pass():
    mesh = plsc.VectorSubcoreMesh(
        core_axis_name="c", subcore_axis_name="s", num_cores=2, num_subcores=16
    )
    return pl.kernel(
        _sc_body,
        out_type=jax.ShapeDtypeStruct((_NW, 512), jnp.float32),
        mesh=mesh,
        compiler_params=pltpu.CompilerParams(
            needs_layout_passes=False, use_tc_tiling_on_sc=True
        ),
        scratch_types=[
            pltpu.VMEM((_ROWS, 512), jnp.float32),
            pltpu.VMEM((_ROWS, 512), jnp.float32),
            pltpu.VMEM((_ROWS, 512), jnp.float32),
            pltpu.VMEM((_ROWS, 512), jnp.float32),
            pltpu.VMEM((_ROWS, 512), jnp.float32),
            pltpu.VMEM((_ROWS, 512), jnp.float32),
            pltpu.VMEM((512,), jnp.float32),
            pltpu.SemaphoreType.DMA,
            pltpu.SemaphoreType.DMA,
        ],
    )


def _tc_body(p_ref, t_ref, o_ref):
    i0 = pl.program_id(0)
    j = pl.program_id(1)

    @pl.when(jnp.logical_and(i0 == 0, j == 0))
    def _():
        o_ref[...] = jnp.zeros_like(o_ref)

    p = p_ref[0, 0]
    a = t_ref[0, 0]
    t = t_ref[0, 1]
    d = (p - a) / (a + 1e-4)
    x = d * d + _MU2
    r = lax.rsqrt(x)
    lt = (x * r - _MU) * t
    g = jnp.abs(d) * r
    b = jnp.minimum(g * 10.0, 9.0).astype(jnp.int32)
    vld = t > 0
    ones_row = jnp.ones((1, _TROWS), jnp.float32)
    for i in range(10):
        m = b == i
        srow = jnp.dot(ones_row, jnp.where(m, lt, 0.0),
                       preferred_element_type=jnp.float32)
        o_ref[pl.ds(i, 1), :] += srow
        mc = jnp.logical_and(m, vld)
        crow = jnp.dot(ones_row, jnp.where(mc, 1.0, 0.0),
                       preferred_element_type=jnp.float32)
        o_ref[pl.ds(16 + i, 1), :] += crow


def _tc_pass(pred, target):
    nj = 512 // _TROWS
    return pl.pallas_call(
        _tc_body,
        grid=(16 - _SCP, nj),
        in_specs=[
            pl.BlockSpec((1, 1, _TROWS, 512), lambda i, j: (i + _SCP, 0, j, 0)),
            pl.BlockSpec((1, 2, _TROWS, 512), lambda i, j: (i + _SCP, 0, j, 0)),
        ],
        out_specs=pl.BlockSpec((32, 512), lambda i, j: (0, 0)),
        out_shape=jax.ShapeDtypeStruct((32, 512), jnp.float32),
    )(pred, target)


def _epi_body(xs_ref, xt_ref, o_ref):
    x = xs_ref[...]
    col = jnp.sum(x, axis=0, keepdims=True)
    wsel = lax.broadcasted_iota(jnp.int32, (32, 512), 1) // 16
    ksel = lax.broadcasted_iota(jnp.int32, (32, 512), 0)
    e = (wsel == ksel).astype(jnp.float32)
    slot = lax.dot_general(e, col, (((1,), (1,)), ((), ())),
                           preferred_element_type=jnp.float32)
    y = xt_ref[...]
    yrow = jnp.sum(y, axis=1, keepdims=True)
    both = slot + yrow
    sb = both[0:16]
    cb = both[16:32]
    tot = jnp.maximum(jnp.sum(cb), 1.0)
    has = cb > 0
    w = jnp.where(has, tot / jnp.maximum(0.25 * cb, 1e-12), 0.0)
    n = jnp.sum(jnp.where(has, 1.0, 0.0))
    o_ref[0, 0] = jnp.sum(w * sb) / jnp.maximum(n, 1.0) / tot


def _epilogue(sc_partials, tc_partials):
    return pl.pallas_call(
        _epi_body,
        out_shape=jax.ShapeDtypeStruct((1, 1), jnp.float32),
        out_specs=pl.BlockSpec(memory_space=pltpu.SMEM),
    )(sc_partials, tc_partials)


def kernel(pred, target):
    sc_partials = _sc_pass()(pred, target)
    tc_partials = _tc_pass(pred, target)
    res = _epilogue(sc_partials, tc_partials)
    return res[0, 0]


# speedup vs baseline: 1.0664x; 1.0664x over previous
import functools

import jax
import jax.numpy as jnp
from jax import lax
from jax.experimental import pallas as pl
from jax.experimental.pallas import tpu as pltpu
from jax.experimental.pallas import tpu_sc as plsc

_MU = 0.02
_MU2 = _MU * _MU
_NW = 32
_ROWS = 16
_SCP = 6
_PERW = _SCP * 512 // _NW
_NCH = _PERW // _ROWS
_U = 4
_TROWS = 256


def _sc_body(pred_hbm, tgt_hbm, out_hbm,
             bpa, b0a, b1a, bpb, b0b, b1b, acc, sema, semb):
    c = lax.axis_index("c")
    s = lax.axis_index("s")
    wid = s * 2 + c

    def zero_body(j, carry):
        acc[pl.ds(j * 16, 16)] = jnp.zeros((16,), jnp.float32)
        return carry

    lax.fori_loop(0, 32, zero_body, 0)

    lane = lax.iota(jnp.int32, 16)
    ones = jnp.ones((16,), jnp.float32)

    def start(ck, bp, b0, b1, sem):
        gr = wid * _PERW + ck * _ROWS
        bat = gr // 512
        r0 = gr % 512
        pltpu.async_copy(pred_hbm.at[bat, 0, pl.ds(r0, _ROWS), :], bp, sem)
        pltpu.async_copy(tgt_hbm.at[bat, 0, pl.ds(r0, _ROWS), :], b0, sem)
        pltpu.async_copy(tgt_hbm.at[bat, 1, pl.ds(r0, _ROWS), :], b1, sem)

    def drain(bp, b0, b1, sem):
        pltpu.make_async_copy(pred_hbm.at[0, 0, pl.ds(0, _ROWS), :], bp, sem).wait()
        pltpu.make_async_copy(pred_hbm.at[0, 0, pl.ds(0, _ROWS), :], b0, sem).wait()
        pltpu.make_async_copy(pred_hbm.at[0, 0, pl.ds(0, _ROWS), :], b1, sem).wait()

    def compute(bp, b0, b1):
        def vec_body(j, c2):
            row = j >> 3
            col = (j & 7) * (16 * _U)
            ps, as_, ts = [], [], []
            for u in range(_U):
                o = col + u * 16
                ps.append(bp[row, pl.ds(o, 16)])
                as_.append(b0[row, pl.ds(o, 16)])
                ts.append(b1[row, pl.ds(o, 16)])
            lts, idxs = [], []
            for u in range(_U):
                p, a, t = ps[u], as_[u], ts[u]
                d = (p - a) / (a + 1e-4)
                x = d * d + _MU2
                i = plsc.bitcast(x, jnp.int32)
                y = plsc.bitcast(0x5F3759DF - (i >> 1), jnp.float32)
                xh = 0.5 * x
                y = y * (1.5 - xh * y * y)
                lts.append((x * y - _MU) * t)
                g = jnp.abs(d) * y
                b = jnp.minimum(g * 10.0, 9.0).astype(jnp.int32)
                idxs.append(b * 16 + lane)
            for u in range(_U):
                plsc.addupdate_scatter(acc, [idxs[u]], lts[u])
                plsc.addupdate_scatter(acc, [idxs[u] + 256], ones,
                                       mask=ts[u] > 0)
            return c2

        lax.fori_loop(0, _ROWS * 512 // (16 * _U), vec_body, 0)

    bufs = ((bpa, b0a, b1a, sema), (bpb, b0b, b1b, semb))
    start(0, *bufs[0])
    for k in range(_NCH):
        cur = bufs[k % 2]
        if k + 1 < _NCH:
            start(k + 1, *bufs[(k + 1) % 2])
        drain(*cur)
        compute(cur[0], cur[1], cur[2])
    pltpu.sync_copy(acc, out_hbm.at[wid])


@functools.lru_cache(maxsize=1)
def _sc_pass():
    mesh = plsc.VectorSubcoreMesh(
        core_axis_name="c", subcore_axis_name="s", num_cores=2, num_subcores=16
    )
    return pl.kernel(
        _sc_body,
        out_type=jax.ShapeDtypeStruct((_NW, 512), jnp.float32),
        mesh=mesh,
        compiler_params=pltpu.CompilerParams(
            needs_layout_passes=False, use_tc_tiling_on_sc=True
        ),
        scratch_types=[
            pltpu.VMEM((_ROWS, 512), jnp.float32),
            pltpu.VMEM((_ROWS, 512), jnp.float32),
            pltpu.VMEM((_ROWS, 512), jnp.float32),
            pltpu.VMEM((_ROWS, 512), jnp.float32),
            pltpu.VMEM((_ROWS, 512), jnp.float32),
            pltpu.VMEM((_ROWS, 512), jnp.float32),
            pltpu.VMEM((512,), jnp.float32),
            pltpu.SemaphoreType.DMA,
            pltpu.SemaphoreType.DMA,
        ],
    )


def _tc_body(p_ref, t_ref, o_ref):
    i0 = pl.program_id(0)
    j = pl.program_id(1)

    @pl.when(jnp.logical_and(i0 == 0, j == 0))
    def _():
        o_ref[...] = jnp.zeros_like(o_ref)

    p = p_ref[0, 0]
    a = t_ref[0, 0]
    t = t_ref[0, 1]
    d = (p - a) / (a + 1e-4)
    x = d * d + _MU2
    r = lax.rsqrt(x)
    lt = (x * r - _MU) * t
    g = jnp.abs(d) * r
    b = jnp.minimum(g * 10.0, 9.0).astype(jnp.int32)
    vld = t > 0
    ones_row = jnp.ones((1, _TROWS), jnp.float32)
    for i in range(10):
        m = b == i
        srow = jnp.dot(ones_row, jnp.where(m, lt, 0.0),
                       preferred_element_type=jnp.float32)
        o_ref[pl.ds(i, 1), :] += srow
        mc = jnp.logical_and(m, vld)
        crow = jnp.dot(ones_row, jnp.where(mc, 1.0, 0.0),
                       preferred_element_type=jnp.float32)
        o_ref[pl.ds(16 + i, 1), :] += crow


def _tc_pass(pred, target):
    nj = 512 // _TROWS
    return pl.pallas_call(
        _tc_body,
        grid=(16 - _SCP, nj),
        in_specs=[
            pl.BlockSpec((1, 1, _TROWS, 512), lambda i, j: (i + _SCP, 0, j, 0)),
            pl.BlockSpec((1, 2, _TROWS, 512), lambda i, j: (i + _SCP, 0, j, 0)),
        ],
        out_specs=pl.BlockSpec((32, 512), lambda i, j: (0, 0)),
        out_shape=jax.ShapeDtypeStruct((32, 512), jnp.float32),
    )(pred, target)


def _epi_body(xs_ref, xt_ref, o_ref):
    x = xs_ref[...]
    col = jnp.sum(x, axis=0, keepdims=True)
    wsel = lax.broadcasted_iota(jnp.int32, (32, 512), 1) // 16
    ksel = lax.broadcasted_iota(jnp.int32, (32, 512), 0)
    e = (wsel == ksel).astype(jnp.float32)
    slot = lax.dot_general(e, col, (((1,), (1,)), ((), ())),
                           preferred_element_type=jnp.float32)
    y = xt_ref[...]
    yrow = jnp.sum(y, axis=1, keepdims=True)
    both = slot + yrow
    sb = both[0:16]
    cb = both[16:32]
    tot = jnp.maximum(jnp.sum(cb), 1.0)
    has = cb > 0
    w = jnp.where(has, tot / jnp.maximum(0.25 * cb, 1e-12), 0.0)
    n = jnp.sum(jnp.where(has, 1.0, 0.0))
    o_ref[0, 0] = jnp.sum(w * sb) / jnp.maximum(n, 1.0) / tot


def _epilogue(sc_partials, tc_partials):
    return pl.pallas_call(
        _epi_body,
        out_shape=jax.ShapeDtypeStruct((1, 1), jnp.float32),
        out_specs=pl.BlockSpec(memory_space=pltpu.SMEM),
    )(sc_partials, tc_partials)


def kernel(pred, target):
    sc_partials = _sc_pass()(pred, target)
    tc_partials = _tc_pass(pred, target)
    res = _epilogue(sc_partials, tc_partials)
    return res[0, 0]


# speedup vs baseline: 1.0757x; 1.0087x over previous
import functools

import jax
import jax.numpy as jnp
from jax import lax
from jax.experimental import pallas as pl
from jax.experimental.pallas import tpu as pltpu
from jax.experimental.pallas import tpu_sc as plsc

_MU = 0.02
_MU2 = _MU * _MU
_NW = 32
_ROWS = 16
_SCP = 6
_PERW = _SCP * 512 // _NW
_NCH = _PERW // _ROWS
_U = 4
_TROWS = 512


def _sc_body(pred_hbm, tgt_hbm, out_hbm,
             bpa, b0a, b1a, bpb, b0b, b1b, acc, sema, semb):
    c = lax.axis_index("c")
    s = lax.axis_index("s")
    wid = s * 2 + c

    def zero_body(j, carry):
        acc[pl.ds(j * 16, 16)] = jnp.zeros((16,), jnp.float32)
        return carry

    lax.fori_loop(0, 32, zero_body, 0)

    lane = lax.iota(jnp.int32, 16)
    ones = jnp.ones((16,), jnp.float32)

    def start(ck, bp, b0, b1, sem):
        gr = wid * _PERW + ck * _ROWS
        bat = gr // 512
        r0 = gr % 512
        pltpu.async_copy(pred_hbm.at[bat, 0, pl.ds(r0, _ROWS), :], bp, sem)
        pltpu.async_copy(tgt_hbm.at[bat, 0, pl.ds(r0, _ROWS), :], b0, sem)
        pltpu.async_copy(tgt_hbm.at[bat, 1, pl.ds(r0, _ROWS), :], b1, sem)

    def drain(bp, b0, b1, sem):
        pltpu.make_async_copy(pred_hbm.at[0, 0, pl.ds(0, _ROWS), :], bp, sem).wait()
        pltpu.make_async_copy(pred_hbm.at[0, 0, pl.ds(0, _ROWS), :], b0, sem).wait()
        pltpu.make_async_copy(pred_hbm.at[0, 0, pl.ds(0, _ROWS), :], b1, sem).wait()

    def compute(bp, b0, b1):
        def vec_body(j, c2):
            row = j >> 3
            col = (j & 7) * (16 * _U)
            ps, as_, ts = [], [], []
            for u in range(_U):
                o = col + u * 16
                ps.append(bp[row, pl.ds(o, 16)])
                as_.append(b0[row, pl.ds(o, 16)])
                ts.append(b1[row, pl.ds(o, 16)])
            lts, idxs = [], []
            for u in range(_U):
                p, a, t = ps[u], as_[u], ts[u]
                d = (p - a) / (a + 1e-4)
                x = d * d + _MU2
                i = plsc.bitcast(x, jnp.int32)
                y = plsc.bitcast(0x5F3759DF - (i >> 1), jnp.float32)
                xh = 0.5 * x
                y = y * (1.5 - xh * y * y)
                lts.append((x * y - _MU) * t)
                g = jnp.abs(d) * y
                b = jnp.minimum(g * 10.0, 9.0).astype(jnp.int32)
                idxs.append(b * 16 + lane)
            for u in range(_U):
                plsc.addupdate_scatter(acc, [idxs[u]], lts[u])
                plsc.addupdate_scatter(acc, [idxs[u] + 256], ones,
                                       mask=ts[u] > 0)
            return c2

        lax.fori_loop(0, _ROWS * 512 // (16 * _U), vec_body, 0)

    bufs = ((bpa, b0a, b1a, sema), (bpb, b0b, b1b, semb))
    start(0, *bufs[0])
    for k in range(_NCH):
        cur = bufs[k % 2]
        if k + 1 < _NCH:
            start(k + 1, *bufs[(k + 1) % 2])
        drain(*cur)
        compute(cur[0], cur[1], cur[2])
    pltpu.sync_copy(acc, out_hbm.at[wid])


@functools.lru_cache(maxsize=1)
def _sc_pass():
    mesh = plsc.VectorSubcoreMesh(
        core_axis_name="c", subcore_axis_name="s", num_cores=2, num_subcores=16
    )
    return pl.kernel(
        _sc_body,
        out_type=jax.ShapeDtypeStruct((_NW, 512), jnp.float32),
        mesh=mesh,
        compiler_params=pltpu.CompilerParams(
            needs_layout_passes=False, use_tc_tiling_on_sc=True
        ),
        scratch_types=[
            pltpu.VMEM((_ROWS, 512), jnp.float32),
            pltpu.VMEM((_ROWS, 512), jnp.float32),
            pltpu.VMEM((_ROWS, 512), jnp.float32),
            pltpu.VMEM((_ROWS, 512), jnp.float32),
            pltpu.VMEM((_ROWS, 512), jnp.float32),
            pltpu.VMEM((_ROWS, 512), jnp.float32),
            pltpu.VMEM((512,), jnp.float32),
            pltpu.SemaphoreType.DMA,
            pltpu.SemaphoreType.DMA,
        ],
    )


def _tc_body(p_ref, t_ref, o_ref):
    i0 = pl.program_id(0)
    j = pl.program_id(1)

    @pl.when(jnp.logical_and(i0 == 0, j == 0))
    def _():
        o_ref[...] = jnp.zeros_like(o_ref)

    p = p_ref[0, 0]
    a = t_ref[0, 0]
    t = t_ref[0, 1]
    d = (p - a) / (a + 1e-4)
    x = d * d + _MU2
    r = lax.rsqrt(x)
    lt = (x * r - _MU) * t
    g = jnp.abs(d) * r
    b = jnp.minimum(g * 10.0, 9.0).astype(jnp.int32)
    vld = t > 0
    ones_row = jnp.ones((1, _TROWS), jnp.float32)
    for i in range(10):
        m = b == i
        srow = jnp.dot(ones_row, jnp.where(m, lt, 0.0),
                       preferred_element_type=jnp.float32)
        o_ref[pl.ds(i, 1), :] += srow
        mc = jnp.logical_and(m, vld)
        crow = jnp.dot(ones_row, jnp.where(mc, 1.0, 0.0),
                       preferred_element_type=jnp.float32)
        o_ref[pl.ds(16 + i, 1), :] += crow


def _tc_pass(pred, target):
    nj = 512 // _TROWS
    return pl.pallas_call(
        _tc_body,
        grid=(16 - _SCP, nj),
        in_specs=[
            pl.BlockSpec((1, 1, _TROWS, 512), lambda i, j: (i + _SCP, 0, j, 0)),
            pl.BlockSpec((1, 2, _TROWS, 512), lambda i, j: (i + _SCP, 0, j, 0)),
        ],
        out_specs=pl.BlockSpec((32, 512), lambda i, j: (0, 0)),
        out_shape=jax.ShapeDtypeStruct((32, 512), jnp.float32),
    )(pred, target)


def _epi_body(xs_ref, xt_ref, o_ref):
    x = xs_ref[...]
    col = jnp.sum(x, axis=0, keepdims=True)
    wsel = lax.broadcasted_iota(jnp.int32, (32, 512), 1) // 16
    ksel = lax.broadcasted_iota(jnp.int32, (32, 512), 0)
    e = (wsel == ksel).astype(jnp.float32)
    slot = lax.dot_general(e, col, (((1,), (1,)), ((), ())),
                           preferred_element_type=jnp.float32)
    y = xt_ref[...]
    yrow = jnp.sum(y, axis=1, keepdims=True)
    both = slot + yrow
    sb = both[0:16]
    cb = both[16:32]
    tot = jnp.maximum(jnp.sum(cb), 1.0)
    has = cb > 0
    w = jnp.where(has, tot / jnp.maximum(0.25 * cb, 1e-12), 0.0)
    n = jnp.sum(jnp.where(has, 1.0, 0.0))
    o_ref[0, 0] = jnp.sum(w * sb) / jnp.maximum(n, 1.0) / tot


def _epilogue(sc_partials, tc_partials):
    return pl.pallas_call(
        _epi_body,
        out_shape=jax.ShapeDtypeStruct((1, 1), jnp.float32),
        out_specs=pl.BlockSpec(memory_space=pltpu.SMEM),
    )(sc_partials, tc_partials)


def kernel(pred, target):
    sc_partials = _sc_pass()(pred, target)
    tc_partials = _tc_pass(pred, target)
    res = _epilogue(sc_partials, tc_partials)
    return res[0, 0]
